# baseline (speedup 1.0000x reference)
import jax
import jax.numpy as jnp
from jax import lax
from jax.experimental import pallas as pl
from jax.experimental.pallas import tpu as pltpu

F = 128
NUM_MESSAGES = 3
ATOM_CLASSES = 64
BOND_CLASSES = 32
NUM_MOLS = 13
B = 2
T = 2
NT = NUM_MOLS + 1
G = T * B
N_ATOMS = 512
N_BONDS = 1024

f32 = jnp.float32
bf16 = jnp.bfloat16
DG0 = (((0,), (0,)), ((), ()))

_BLK_NAMES = ('Wgu1', 'bgu1', 'Wgu2', 'bgu2', 'We1', 'be1', 'We2', 'be2',
              'Wn1', 'bn1', 'Wn2', 'bn2', 'Wnp1', 'bnp1', 'Wnp2', 'bnp2')


def _hilo_cols(a):
    hi = a.astype(bf16)
    lo = (a - hi.astype(f32)).astype(bf16)
    return jnp.concatenate([hi, lo], axis=1)


def _dot(a, b):
    return jnp.dot(a, b, preferred_element_type=f32)


def _halves(x):
    return x[:, 0:F] + x[:, F:2 * F]


def _tower_body(*refs):
    (atoms_ref, bonds_ref, conn0_ref, conn1_ref, mf_ref, x_ref,
     aemb_ref, bemb_ref, wg0_ref, bg0_ref) = refs[:10]
    blk = [dict(zip(_BLK_NAMES, refs[10 + 16 * i:10 + 16 * (i + 1)]))
           for i in range(NUM_MESSAGES)]
    wp_ref, bp_ref, out_ref, acc_ref = refs[58], refs[59], refs[60], refs[61]

    m = pl.program_id(0)

    @pl.when(m == 0)
    def _init():
        acc_ref[...] = jnp.zeros_like(acc_ref)

    E2 = 2 * N_BONDS

    iota_g = lax.broadcasted_iota(jnp.int32, (N_ATOMS, E2), 0)
    iota_a = lax.broadcasted_iota(jnp.int32, (ATOM_CLASSES, N_ATOMS), 0)
    iota_b = lax.broadcasted_iota(jnp.int32, (BOND_CLASSES, N_BONDS), 0)

    ohg = []
    a_states = []
    b_states = []
    for t in range(T):
        for b in range(B):
            c0_row = conn0_ref[t, b:b + 1, :]
            c1_row = conn1_ref[t, b:b + 1, :]
            conn_cat = jnp.concatenate([c1_row, c0_row], axis=1)
            ohg.append((conn_cat == iota_g).astype(bf16))

            atom_row = atoms_ref[t, b:b + 1, :]
            oha = (atom_row == iota_a).astype(bf16)
            a_states.append(_halves(lax.dot_general(
                oha, aemb_ref[...], DG0, preferred_element_type=f32)))
            bond_row = bonds_ref[t, b:b + 1, :]
            ohb = (bond_row == iota_b).astype(bf16)
            b_states.append(_halves(lax.dot_general(
                ohb, bemb_ref[...], DG0, preferred_element_type=f32)))

    atom_state = jnp.concatenate(a_states, axis=0)
    bond_state = jnp.concatenate(b_states, axis=0)

    mf = mf_ref[...].reshape(G, 2)
    gs = jax.nn.relu(_dot(mf, wg0_ref[...]) + bg0_ref[...])

    for i in range(NUM_MESSAGES):
        bk = blk[i]
        g = jnp.concatenate(
            [jnp.mean(atom_state[k * N_ATOMS:(k + 1) * N_ATOMS], axis=0,
                      keepdims=True) for k in range(G)], axis=0)
        g = jax.nn.relu(_dot(g, bk['Wgu1'][...]) + bk['bgu1'][...])
        g = _dot(g, bk['Wgu2'][...]) + bk['bgu2'][...]
        gs = gs + g

        a_hl = _hilo_cols(atom_state)
        srcs, tgts = [], []
        for k in range(G):
            st = _halves(lax.dot_general(
                ohg[k], a_hl[k * N_ATOMS:(k + 1) * N_ATOMS], DG0,
                preferred_element_type=f32))
            srcs.append(st[0:N_BONDS])
            tgts.append(st[N_BONDS:])
        src = jnp.concatenate(srcs, axis=0)
        tgt = jnp.concatenate(tgts, axis=0)

        we1 = bk['We1']
        gterm_e = _dot(gs, we1[3 * F:4 * F]) + bk['be1'][...]
        gterm_e = jnp.broadcast_to(
            gterm_e[:, None, :], (G, N_BONDS, 2 * F)).reshape(G * N_BONDS,
                                                              2 * F)
        h = jax.nn.relu(_dot(bond_state, we1[0:F])
                        + _dot(src, we1[F:2 * F])
                        + _dot(tgt, we1[2 * F:3 * F])
                        + gterm_e)
        bond_state = bond_state + _dot(h, bk['We2'][...]) + bk['be2'][...]

        wn1 = bk['Wn1']
        gterm_n = _dot(gs, wn1[2 * F:3 * F]) + bk['bn1'][...]
        gterm_n = jnp.broadcast_to(
            gterm_n[:, None, :], (G, N_BONDS, 2 * F)).reshape(G * N_BONDS,
                                                              2 * F)
        hm = jax.nn.relu(_dot(src, wn1[0:F])
                         + _dot(bond_state, wn1[F:2 * F])
                         + gterm_n)
        messages = _dot(hm, bk['Wn2'][...]) + bk['bn2'][...]

        m_hl = _hilo_cols(messages)
        reds = []
        for k in range(G):
            mb = m_hl[k * N_BONDS:(k + 1) * N_BONDS]
            reds.append(_halves(_dot(ohg[k][:, N_BONDS:], mb)))
        reduced = jnp.concatenate(reds, axis=0)

        na = _dot(jax.nn.relu(_dot(reduced, bk['Wnp1'][...]) + bk['bnp1'][...]),
                  bk['Wnp2'][...]) + bk['bnp2'][...]
        atom_state = atom_state + na

    pred = _dot(gs, wp_ref[...]) + bp_ref[...]
    xcol = jnp.concatenate(
        [jnp.full((1, 1), x_ref[b, T * m + t], f32)
         for t in range(T) for b in range(B)], axis=0)
    acc_ref[...] = acc_ref[...] + xcol * pred

    @pl.when(m == NT // T - 1)
    def _fin():
        a = acc_ref[...]
        ab = a[0:B] + a[B:2 * B]
        out_ref[...] = ab[:, 0:1] * ab[:, 2:3] + ab[:, 1:2] * ab[:, 3:4]


def kernel(atoms, bonds, connectivity, mol_features, X, params):
    blocks = params['blocks']

    def pad(x):
        return jnp.concatenate([x, x[:1]], axis=0)

    conn0 = pad(connectivity[..., 0])
    conn1 = pad(connectivity[..., 1])
    Xp = jnp.concatenate([X, jnp.zeros((B, 1), f32)], axis=1)

    def hilo_w(w):
        hi = w.astype(bf16)
        lo = (w - hi.astype(f32)).astype(bf16)
        return jnp.concatenate([hi, lo], axis=1)

    inputs = [
        pad(atoms), pad(bonds), conn0, conn1, pad(mol_features), Xp,
        hilo_w(params['atom_emb']), hilo_w(params['bond_emb']),
        params['Wg0'], params['bg0'].reshape(1, F),
    ]
    for blk in blocks:
        for name in _BLK_NAMES:
            w = blk[name]
            inputs.append(w.reshape(1, -1) if w.ndim == 1 else w)
    inputs += [params['Wp'], params['bp'].reshape(1, 4)]

    def bspec(shape, blocked_lead=False):
        if blocked_lead:
            blk_shape = (T,) + shape[1:]
            return pl.BlockSpec(blk_shape,
                                lambda m: (m,) + (0,) * (len(shape) - 1))
        return pl.BlockSpec(shape, lambda m: (0,) * len(shape))

    in_specs = [
        bspec(inputs[0].shape, True), bspec(inputs[1].shape, True),
        bspec(conn0.shape, True), bspec(conn1.shape, True),
        bspec(inputs[4].shape, True),
        pl.BlockSpec(memory_space=pltpu.SMEM),
    ] + [bspec(x.shape) for x in inputs[6:]]

    out = pl.pallas_call(
        _tower_body,
        grid=(NT // T,),
        in_specs=in_specs,
        out_specs=pl.BlockSpec((B, 1), lambda m: (0, 0)),
        out_shape=jax.ShapeDtypeStruct((B, 1), jnp.float32),
        scratch_shapes=[pltpu.VMEM((G, 4), jnp.float32)],
    )(*inputs)
    return out.reshape(B)
